# baseline (speedup 1.0000x reference)
import functools

import jax
import jax.numpy as jnp
from jax import lax
from jax.experimental import pallas as pl
from jax.experimental.pallas import tpu as pltpu
from jax.experimental.pallas import tpu_sc as plsc


def _topk_body(x1_ref, x2t_ref, *rest, n2, tile, batch, c2):
    if len(rest) == 2:
        w_ref, a_ref = rest
    else:
        if len(rest) == 5:
            flat_ref, f1_ref, w_ref, a_ref, out_ref = rest
        else:
            flat_ref, f1_ref, _prev, w_ref, a_ref, out_ref = rest
        _asm_body(flat_ref, f1_ref, out_ref, c2=c2)
    x1 = x1_ref[0]
    x2 = x2t_ref[0]

    e0 = x2[:, 0:1] - x1[0:1, :]
    e1 = x2[:, 1:2] - x1[1:2, :]
    e2 = x2[:, 2:3] - x1[2:3, :]
    d = (e0 * e0 + e1 * e1) + e2 * e2

    fio = lax.broadcasted_iota(jnp.int32, (n2, tile), 0).astype(jnp.float32)
    dists, afs = [], []
    for p in range(3):
        m = jnp.min(d, axis=0, keepdims=True)
        af = jnp.min(jnp.where(d == m, fio, 1e9), axis=0, keepdims=True)
        if p < 2:
            d = jnp.where(fio == af, jnp.inf, d)
        dists.append(m)
        afs.append(af)

    rs = [1.0 / jnp.maximum(m, 1e-10) for m in dists]
    norm = (rs[0] + rs[1]) + rs[2]

    w_ref[0] = jnp.concatenate([rs[0] / norm, rs[1] / norm, rs[2] / norm], 0)
    a_ref[0] = jnp.concatenate(afs, 0).astype(jnp.int32) + batch * n2


def _topk(xyz1b, xyz2tb, n2, tile, batch, asm=None):
    n1 = xyz1b.shape[2]
    grid = (1, n1 // tile)
    in_specs = [
        pl.BlockSpec((1, 3, tile), lambda ib, it: (ib, 0, it)),
        pl.BlockSpec((1, n2, 3), lambda ib, it: (ib, 0, 0)),
    ]
    out_specs = [
        pl.BlockSpec((1, 3, tile), lambda ib, it: (ib, 0, it)),
        pl.BlockSpec((1, 3, tile), lambda ib, it: (ib, 0, it)),
    ]
    out_shape = [
        jax.ShapeDtypeStruct((1, 3, n1), jnp.float32),
        jax.ShapeDtypeStruct((1, 3, n1), jnp.int32),
    ]
    args = [xyz1b, xyz2tb]
    kwargs = {}
    c2 = 0
    if asm is not None:
        flat_prev, f1_prev, out_prev, asm_ib, c1, c2, nb = asm
        in_specs += [
            pl.BlockSpec((tile, c2), lambda ib, it: (it, 0)),
            pl.BlockSpec((1, c1, tile), lambda ib, it: (0, 0, it)),
        ]
        out_specs.append(
            pl.BlockSpec((1, c1 + c2, tile),
                         lambda ib, it, _ab=asm_ib: (_ab, 0, it)))
        out_shape.append(
            jax.ShapeDtypeStruct((nb, c1 + c2, n1), jnp.float32))
        args += [flat_prev, f1_prev]
        if out_prev is not None:
            in_specs.append(pl.BlockSpec(memory_space=pl.ANY))
            args.append(out_prev)
            kwargs["input_output_aliases"] = {4: 2}
    res = pl.pallas_call(
        functools.partial(_topk_body, n2=n2, tile=tile, batch=batch, c2=c2),
        grid=grid,
        in_specs=in_specs,
        out_specs=out_specs,
        out_shape=out_shape,
        **kwargs,
    )(*args)
    return res


_G = 64


def _sc_combine(nq, c2, widx, wts, f2r):
    info = plsc.get_sparse_core_info()
    nw = info.num_cores * info.num_subcores
    per_w = nq // nw
    mesh = plsc.VectorSubcoreMesh(core_axis_name="c", subcore_axis_name="s")

    @functools.partial(
        pl.kernel, mesh=mesh,
        out_type=jax.ShapeDtypeStruct((nq, c2), jnp.float32),
        scratch_types=[
            pltpu.VMEM((per_w,), jnp.int32),
            pltpu.VMEM((per_w,), jnp.int32),
            pltpu.VMEM((per_w,), jnp.int32),
            pltpu.VMEM((per_w,), jnp.float32),
            pltpu.VMEM((per_w,), jnp.float32),
            pltpu.VMEM((per_w,), jnp.float32),
            pltpu.VMEM((3 * _G, c2), jnp.float32),
            pltpu.VMEM((_G, c2), jnp.float32),
            pltpu.SemaphoreType.DMA,
        ],
    )
    def k(a_hbm, w_hbm, f2r_hbm, out_hbm,
          i0_v, i1_v, i2_v, w0_v, w1_v, w2_v, rows_v, outb_v, sem):
        wid = lax.axis_index("s") * info.num_cores + lax.axis_index("c")
        q0 = wid * per_w
        for kk, (iv, wvk) in enumerate(
                [(i0_v, w0_v), (i1_v, w1_v), (i2_v, w2_v)]):
            off = kk * nq + q0
            pltpu.sync_copy(a_hbm.at[pl.ds(off, per_w)], iv)
            pltpu.sync_copy(w_hbm.at[pl.ds(off, per_w)], wvk)

        def chunk(ci, _):
            cq = ci * _G
            cps = [
                pltpu.async_copy(
                    f2r_hbm.at[iv.at[pl.ds(cq, _G)]],
                    rows_v.at[pl.ds(kk * _G, _G)], sem)
                for kk, iv in enumerate([i0_v, i1_v, i2_v])
            ]
            for cp in cps:
                cp.wait()

            def gbody(gi, _):
                g0 = gi * 16
                wv0 = w0_v[pl.ds(cq + g0, 16)]
                wv1 = w1_v[pl.ds(cq + g0, 16)]
                wv2 = w2_v[pl.ds(cq + g0, 16)]
                for q in range(16):
                    w0, w1, w2 = wv0[q], wv1[q], wv2[q]
                    for cc in range(c2 // 16):
                        sl = pl.ds(cc * 16, 16)
                        outb_v[g0 + q, sl] = (rows_v[g0 + q, sl] * w0
                                              + rows_v[_G + g0 + q, sl] * w1) \
                                              + rows_v[2 * _G + g0 + q, sl] * w2
                return 0

            lax.fori_loop(0, _G // 16, gbody, 0)
            pltpu.sync_copy(outb_v, out_hbm.at[pl.ds(q0 + cq, _G)])
            return 0

        lax.fori_loop(0, per_w // _G, chunk, 0)

    return k(widx, wts, f2r)


def _asm_body(flat_ref, f1_ref, out_ref, *, c2):
    out_ref[0, :c2, :] = jnp.transpose(flat_ref[...], (1, 0))
    out_ref[0, c2:, :] = f1_ref[0]


def _assemble(out_prev, flat, f1b, ib, c1, c2, n1, tile):
    def body(flat_ref, f1_ref, prev_ref, out_ref):
        del prev_ref
        _asm_body(flat_ref, f1_ref, out_ref, c2=c2)

    return pl.pallas_call(
        body,
        grid=(n1 // tile,),
        in_specs=[
            pl.BlockSpec((tile, c2), lambda it: (it, 0)),
            pl.BlockSpec((1, c1, tile), lambda it: (0, 0, it)),
            pl.BlockSpec(memory_space=pl.ANY),
        ],
        out_specs=pl.BlockSpec((1, c1 + c2, tile), lambda it, _ib=ib: (_ib, 0, it)),
        out_shape=jax.ShapeDtypeStruct(out_prev.shape, jnp.float32),
        input_output_aliases={2: 0},
    )(flat, f1b, out_prev)


def kernel(xyz1, xyz2, features1, features2):
    b, _, n1 = xyz1.shape
    n2 = xyz2.shape[2]
    c1 = features1.shape[1]
    c2 = features2.shape[1]
    tile = min(1024, n1)

    xyz2t = jnp.transpose(xyz2, (0, 2, 1))
    f2r = jnp.transpose(features2, (0, 2, 1)).reshape(b * n2, c2)

    flats = []
    for ib in range(b):
        wts, widx = _topk(xyz1[ib:ib + 1], xyz2t[ib:ib + 1], n2, tile, ib)
        flats.append(_sc_combine(n1, c2, widx.reshape(-1), wts.reshape(-1), f2r))

    def asm_all(*refs):
        flat_refs, f1_ref, out_ref = refs[:b], refs[b], refs[b + 1]
        for ib, fr in enumerate(flat_refs):
            out_ref[ib, :c2, :] = jnp.transpose(fr[...], (1, 0))
        out_ref[:, c2:, :] = f1_ref[...]

    return pl.pallas_call(
        asm_all,
        grid=(n1 // tile,),
        in_specs=[pl.BlockSpec((tile, c2), lambda it: (it, 0))] * b
        + [pl.BlockSpec((b, c1, tile), lambda it: (0, 0, it))],
        out_specs=pl.BlockSpec((b, c1 + c2, tile), lambda it: (0, 0, it)),
        out_shape=jax.ShapeDtypeStruct((b, c1 + c2, n1), jnp.float32),
    )(*flats, features1)


# speedup vs baseline: 1.0131x; 1.0131x over previous
import functools

import jax
import jax.numpy as jnp
from jax import lax
from jax.experimental import pallas as pl
from jax.experimental.pallas import tpu as pltpu
from jax.experimental.pallas import tpu_sc as plsc


def _topk_body(x1_ref, x2t_ref, w_ref, a_ref, *, n2, tile, batch):
    x1 = x1_ref[0]
    x2 = x2t_ref[0]

    e0 = x2[:, 0:1] - x1[0:1, :]
    e1 = x2[:, 1:2] - x1[1:2, :]
    e2 = x2[:, 2:3] - x1[2:3, :]
    d = (e0 * e0 + e1 * e1) + e2 * e2

    fio = lax.broadcasted_iota(jnp.int32, (n2, tile), 0).astype(jnp.float32)
    dists, afs = [], []
    for p in range(3):
        m = jnp.min(d, axis=0, keepdims=True)
        af = jnp.min(jnp.where(d == m, fio, 1e9), axis=0, keepdims=True)
        if p < 2:
            d = jnp.where(fio == af, jnp.inf, d)
        dists.append(m)
        afs.append(af)

    rs = [1.0 / jnp.maximum(m, 1e-10) for m in dists]
    norm = (rs[0] + rs[1]) + rs[2]

    w_ref[0] = jnp.concatenate([rs[0] / norm, rs[1] / norm, rs[2] / norm], 0)
    a_ref[0] = jnp.concatenate(afs, 0).astype(jnp.int32) + batch * n2


def _topk(xyz1b, xyz2tb, n2, tile, batch):
    n1 = xyz1b.shape[2]
    grid = (1, n1 // tile)
    return pl.pallas_call(
        functools.partial(_topk_body, n2=n2, tile=tile, batch=batch),
        grid=grid,
        in_specs=[
            pl.BlockSpec((1, 3, tile), lambda ib, it: (ib, 0, it)),
            pl.BlockSpec((1, n2, 3), lambda ib, it: (ib, 0, 0)),
        ],
        out_specs=[
            pl.BlockSpec((1, 3, tile), lambda ib, it: (ib, 0, it)),
            pl.BlockSpec((1, 3, tile), lambda ib, it: (ib, 0, it)),
        ],
        out_shape=[
            jax.ShapeDtypeStruct((1, 3, n1), jnp.float32),
            jax.ShapeDtypeStruct((1, 3, n1), jnp.int32),
        ],
    )(xyz1b, xyz2tb)


_G = 64


def _sc_combine(nq, c2, widx, wts, f2r):
    info = plsc.get_sparse_core_info()
    nw = info.num_cores * info.num_subcores
    per_w = nq // nw
    mesh = plsc.VectorSubcoreMesh(core_axis_name="c", subcore_axis_name="s")

    @functools.partial(
        pl.kernel, mesh=mesh,
        out_type=jax.ShapeDtypeStruct((nq, c2), jnp.float32),
        scratch_types=[
            pltpu.VMEM((per_w,), jnp.int32),
            pltpu.VMEM((per_w,), jnp.int32),
            pltpu.VMEM((per_w,), jnp.int32),
            pltpu.VMEM((per_w,), jnp.float32),
            pltpu.VMEM((per_w,), jnp.float32),
            pltpu.VMEM((per_w,), jnp.float32),
            pltpu.VMEM((3 * _G, c2), jnp.float32),
            pltpu.VMEM((_G, c2), jnp.float32),
            pltpu.SemaphoreType.DMA,
        ],
    )
    def k(a_hbm, w_hbm, f2r_hbm, out_hbm,
          i0_v, i1_v, i2_v, w0_v, w1_v, w2_v, rows_v, outb_v, sem):
        wid = lax.axis_index("s") * info.num_cores + lax.axis_index("c")
        q0 = wid * per_w
        for kk, (iv, wvk) in enumerate(
                [(i0_v, w0_v), (i1_v, w1_v), (i2_v, w2_v)]):
            off = kk * nq + q0
            pltpu.sync_copy(a_hbm.at[pl.ds(off, per_w)], iv)
            pltpu.sync_copy(w_hbm.at[pl.ds(off, per_w)], wvk)

        def chunk(ci, _):
            cq = ci * _G
            cps = [
                pltpu.async_copy(
                    f2r_hbm.at[iv.at[pl.ds(cq, _G)]],
                    rows_v.at[pl.ds(kk * _G, _G)], sem)
                for kk, iv in enumerate([i0_v, i1_v, i2_v])
            ]
            for cp in cps:
                cp.wait()

            def gbody(gi, _):
                g0 = gi * 16
                wv0 = w0_v[pl.ds(cq + g0, 16)]
                wv1 = w1_v[pl.ds(cq + g0, 16)]
                wv2 = w2_v[pl.ds(cq + g0, 16)]
                for q in range(16):
                    w0, w1, w2 = wv0[q], wv1[q], wv2[q]
                    for cc in range(c2 // 16):
                        sl = pl.ds(cc * 16, 16)
                        outb_v[g0 + q, sl] = (rows_v[g0 + q, sl] * w0
                                              + rows_v[_G + g0 + q, sl] * w1) \
                                              + rows_v[2 * _G + g0 + q, sl] * w2
                return 0

            lax.fori_loop(0, _G // 16, gbody, 0)
            pltpu.sync_copy(outb_v, out_hbm.at[pl.ds(q0 + cq, _G)])
            return 0

        lax.fori_loop(0, per_w // _G, chunk, 0)

    return k(widx, wts, f2r)


def _asm_body(flat_ref, f1_ref, out_ref, *, c2):
    out_ref[0, :c2, :] = jnp.transpose(flat_ref[...], (1, 0))
    out_ref[0, c2:, :] = f1_ref[0]


def _assemble(out_prev, flat, f1b, ib, c1, c2, n1, tile):
    def body(flat_ref, f1_ref, prev_ref, out_ref):
        del prev_ref
        _asm_body(flat_ref, f1_ref, out_ref, c2=c2)

    return pl.pallas_call(
        body,
        grid=(n1 // tile,),
        in_specs=[
            pl.BlockSpec((tile, c2), lambda it: (it, 0)),
            pl.BlockSpec((1, c1, tile), lambda it: (0, 0, it)),
            pl.BlockSpec(memory_space=pl.ANY),
        ],
        out_specs=pl.BlockSpec((1, c1 + c2, tile), lambda it, _ib=ib: (_ib, 0, it)),
        out_shape=jax.ShapeDtypeStruct(out_prev.shape, jnp.float32),
        input_output_aliases={2: 0},
    )(flat, f1b, out_prev)


def kernel(xyz1, xyz2, features1, features2):
    b, _, n1 = xyz1.shape
    n2 = xyz2.shape[2]
    c1 = features1.shape[1]
    c2 = features2.shape[1]
    tile = min(1024, n1)

    xyz2t = jnp.transpose(xyz2, (0, 2, 1))
    f2r = jnp.transpose(features2, (0, 2, 1)).reshape(b * n2, c2)

    out = None
    flats = []
    for ib in range(b):
        wts, widx = _topk(xyz1[ib:ib + 1], xyz2t[ib:ib + 1], n2, tile, ib)
        flats.append(_sc_combine(n1, c2, widx.reshape(-1), wts.reshape(-1), f2r))
    for ib in range(b):
        if out is None:
            out = pl.pallas_call(
                functools.partial(_asm_body, c2=c2),
                grid=(n1 // tile,),
                in_specs=[
                    pl.BlockSpec((tile, c2), lambda it: (it, 0)),
                    pl.BlockSpec((1, c1, tile), lambda it: (0, 0, it)),
                ],
                out_specs=pl.BlockSpec(
                    (1, c1 + c2, tile), lambda it, _ib=ib: (_ib, 0, it)),
                out_shape=jax.ShapeDtypeStruct((b, c1 + c2, n1), jnp.float32),
            )(flats[ib], features1[ib:ib + 1])
        else:
            out = _assemble(out, flats[ib], features1[ib:ib + 1],
                            ib, c1, c2, n1, tile)
    return out
